# baseline (speedup 1.0000x reference)
import functools

import jax
import jax.numpy as jnp
from jax import lax
from jax.experimental import pallas as pl
from jax.experimental.pallas import tpu as pltpu
from jax.experimental.pallas import tpu_sc as plsc

CHUNK = 256
NBUF = 12
DEPTH = 8
OUTB = 4


@functools.partial(jax.jit, static_argnums=(2, 3, 4))
def _sc_gather(idx2, table, nch, num_workers, embed_dim):
    n = idx2.shape[0] * CHUNK
    mesh = plsc.VectorSubcoreMesh(core_axis_name="c", subcore_axis_name="s")
    nc = mesh.num_cores
    nbatch = nch // OUTB
    nbring = NBUF // OUTB
    assert nch % OUTB == 0 and NBUF % OUTB == 0 and DEPTH + OUTB <= NBUF

    @functools.partial(
        pl.kernel,
        out_type=jax.ShapeDtypeStruct((n, embed_dim), jnp.float32),
        mesh=mesh,
        scratch_types=(
            [pltpu.VMEM((nch, CHUNK), jnp.int32)]
            + [pltpu.VMEM((NBUF * CHUNK, embed_dim), jnp.float32)]
            + [pltpu.SemaphoreType.DMA] * (NBUF + nbring)
        ),
        compiler_params=pltpu.CompilerParams(use_tc_tiling_on_sc=False),
    )
    def run(idx_hbm, table_hbm, out_hbm, idxv, rowsv, *sems):
        gsem = sems[:NBUF]
        osem = sems[NBUF:]
        wid = lax.axis_index("s") * nc + lax.axis_index("c")
        cbase = wid * nch
        rbase = cbase * CHUNK
        pltpu.sync_copy(idx_hbm.at[pl.ds(cbase, nch)], idxv)

        def start_gather(c):
            b = c % NBUF
            pltpu.async_copy(
                table_hbm.at[idxv.at[c]],
                rowsv.at[pl.ds(b * CHUNK, CHUNK)],
                gsem[b],
            )

        def wait_gather(c):
            b = c % NBUF
            pltpu.make_async_copy(
                table_hbm.at[idxv.at[c]],
                rowsv.at[pl.ds(b * CHUNK, CHUNK)],
                gsem[b],
            ).wait()

        def out_copy(bt):
            p = (bt % nbring) * OUTB * CHUNK
            return pltpu.make_async_copy(
                rowsv.at[pl.ds(p, OUTB * CHUNK)],
                out_hbm.at[pl.ds(rbase + bt * OUTB * CHUNK, OUTB * CHUNK)],
                osem[bt % nbring],
            )

        for c in range(DEPTH):
            start_gather(c)
        for c in range(nch):
            g = c + DEPTH
            if g < nch:
                if g >= NBUF and (g - NBUF) % OUTB == 0:
                    out_copy((g - NBUF) // OUTB).wait()
                start_gather(g)
            wait_gather(c)
            if c % OUTB == OUTB - 1:
                out_copy(c // OUTB).start()
        for bt in range(nbatch - nbring, nbatch):
            out_copy(bt).wait()

    return run(idx2, table)


def kernel(inputs, embeddings):
    b, f = inputs.shape
    v, d = embeddings.shape
    n = b * f
    num_workers = 32
    assert n % (num_workers * CHUNK) == 0
    nch = n // (num_workers * CHUNK)
    idx2 = inputs.astype(jnp.int32).reshape(n // CHUNK, CHUNK)
    out = _sc_gather(idx2, embeddings, nch, num_workers, d)
    return out.reshape(b, f, d)
